# baseline (speedup 1.0000x reference)
import functools

import jax
import jax.numpy as jnp
from jax import lax
from jax.experimental import pallas as pl
from jax.experimental.pallas import tpu as pltpu
from jax.experimental.pallas import tpu_sc as plsc

N = 1_000_000
D = 64
NC = 2
NS = 16
NW = NC * NS
L = 16

CHUNK = 400
MAIN_PER_TILE = 78
PAIRS = MAIN_PER_TILE // 2
ROWS_PER_TILE = CHUNK * MAIN_PER_TILE
MAIN_ROWS = ROWS_PER_TILE * NW
TAIL_CHUNKS = (N - MAIN_ROWS) // CHUNK
GROUPS = CHUNK // L


def _rsqrt16(x):
    xi = plsc.bitcast(x, jnp.int32)
    y = plsc.bitcast(jnp.int32(0x5F3759DF) - (xi >> 1), jnp.float32)
    xh = x * jnp.float32(-0.5)
    for _ in range(3):
        y = y * (jnp.float32(1.5) + xh * y * y)
    return y


def _sc_body(cand_hbm, slots_hbm, scores_hbm, pmax_hbm, pidx_hbm,
             cand_v, in_a, in_b, sc_a, sc_b, mvec, ivec,
             sem_ia, sem_ib, sem_oa, sem_ob):
    c = lax.axis_index("c")
    s = lax.axis_index("s")
    wid = s * NC + c

    pltpu.sync_copy(cand_hbm, cand_v)
    c0 = cand_v[pl.ds(0, L)]
    c1 = cand_v[pl.ds(L, L)]
    c2 = cand_v[pl.ds(2 * L, L)]
    c3 = cand_v[pl.ds(3 * L, L)]
    cn2 = jnp.sum(c0 * c0 + c1 * c1 + c2 * c2 + c3 * c3)
    inv_c = _rsqrt16(jnp.full((L,), jnp.maximum(cn2, jnp.float32(1e-30)),
                              jnp.float32))
    cn = (c0 * inv_c, c1 * inv_c, c2 * inv_c, c3 * inv_c)

    mvec[...] = jnp.full((L,), -jnp.inf, jnp.float32)
    ivec[...] = jnp.zeros((L,), jnp.int32)
    iota = lax.iota(jnp.int32, L)
    row_stride = iota * D

    def compute_chunk(in_v, sc_v, row0):
        def group(g, carry):
            vbase = row_stride + g * (L * D)
            acc_d = jnp.zeros((L,), jnp.float32)
            acc_n = jnp.zeros((L,), jnp.float32)
            for j in range(D):
                w = plsc.load_gather(in_v, [vbase + j])
                acc_d = acc_d + w * cn[j // L][j % L]
                acc_n = acc_n + w * w
            inv = _rsqrt16(jnp.maximum(acc_n, jnp.float32(1e-30)))
            sc16 = acc_d * inv
            sc_v[pl.ds(g * L, L)] = sc16
            idx16 = iota + (row0 + g * L)
            m = mvec[...]
            better = sc16 > m
            mvec[...] = jnp.where(better, sc16, m)
            ivec[...] = jnp.where(better, idx16, ivec[...])
            return carry

        lax.fori_loop(0, GROUPS, group, 0)

    def in_copy(row0, buf, sem):
        return pltpu.make_async_copy(
            slots_hbm.at[pl.ds(row0 * D, CHUNK * D)], buf, sem)

    def out_copy(row0, buf, sem):
        return pltpu.make_async_copy(
            buf, scores_hbm.at[pl.ds(row0, CHUNK)], sem)

    base = wid * ROWS_PER_TILE
    in_copy(base, in_a, sem_ia).start()

    def pair(p, carry):
        r0 = base + (2 * p) * CHUNK
        in_copy(r0, in_a, sem_ia).wait()
        in_copy(r0 + CHUNK, in_b, sem_ib).start()

        @pl.when(p > 0)
        def _():
            out_copy(r0 - 2 * CHUNK, sc_a, sem_oa).wait()

        compute_chunk(in_a, sc_a, r0)
        out_copy(r0, sc_a, sem_oa).start()

        in_copy(r0 + CHUNK, in_b, sem_ib).wait()

        @pl.when(p < PAIRS - 1)
        def _():
            in_copy(r0 + 2 * CHUNK, in_a, sem_ia).start()

        @pl.when(p > 0)
        def _():
            out_copy(r0 - CHUNK, sc_b, sem_ob).wait()

        compute_chunk(in_b, sc_b, r0 + CHUNK)
        out_copy(r0 + CHUNK, sc_b, sem_ob).start()
        return carry

    lax.fori_loop(0, PAIRS, pair, 0)
    out_copy(base + (MAIN_PER_TILE - 2) * CHUNK, sc_a, sem_oa).wait()
    out_copy(base + (MAIN_PER_TILE - 1) * CHUNK, sc_b, sem_ob).wait()

    @pl.when(wid < TAIL_CHUNKS)
    def _():
        row0 = MAIN_ROWS + wid * CHUNK
        pltpu.sync_copy(slots_hbm.at[pl.ds(row0 * D, CHUNK * D)], in_a)
        compute_chunk(in_a, sc_a, row0)
        pltpu.sync_copy(sc_a, scores_hbm.at[pl.ds(row0, CHUNK)])

    pltpu.sync_copy(mvec, pmax_hbm.at[wid])
    pltpu.sync_copy(ivec, pidx_hbm.at[wid])


def _merge_body(pm_ref, pi_ref, o_ref):
    m = pm_ref[...]
    i = pi_ref[...]
    best = jnp.max(m)
    o_ref[0, 0] = jnp.min(jnp.where(m == best, i, jnp.int32(2147483647)))


def _merge(pmax, pidx):
    return pl.pallas_call(
        _merge_body,
        out_shape=jax.ShapeDtypeStruct((1, 1), jnp.int32),
        out_specs=pl.BlockSpec(memory_space=pltpu.SMEM),
    )(pmax, pidx)


@jax.jit
def kernel(candidate, slot_embeds):
    mesh = plsc.VectorSubcoreMesh(core_axis_name="c", subcore_axis_name="s")
    sc_call = pl.kernel(
        _sc_body,
        out_type=[
            jax.ShapeDtypeStruct((N,), jnp.float32),
            jax.ShapeDtypeStruct((NW, L), jnp.float32),
            jax.ShapeDtypeStruct((NW, L), jnp.int32),
        ],
        scratch_types=[
            pltpu.VMEM((D,), jnp.float32),
            pltpu.VMEM((CHUNK * D,), jnp.float32),
            pltpu.VMEM((CHUNK * D,), jnp.float32),
            pltpu.VMEM((CHUNK,), jnp.float32),
            pltpu.VMEM((CHUNK,), jnp.float32),
            pltpu.VMEM((L,), jnp.float32),
            pltpu.VMEM((L,), jnp.int32),
            pltpu.SemaphoreType.DMA,
            pltpu.SemaphoreType.DMA,
            pltpu.SemaphoreType.DMA,
            pltpu.SemaphoreType.DMA,
        ],
        mesh=mesh,
        compiler_params=pltpu.CompilerParams(needs_layout_passes=False),
    )
    scores, pmax, pidx = sc_call(candidate, slot_embeds.reshape(-1))
    best = _merge(pmax, pidx)[0, 0]
    return scores, best
